# baseline (speedup 1.0000x reference)
import jax
import jax.numpy as jnp
from jax import lax
from jax.experimental import pallas as pl
from jax.experimental.pallas import tpu as pltpu
from jax.experimental.pallas import tpu_sc as plsc

VOCAB = 49408
NUM_POS = 77
EMBED_DIM = 768
BATCH = 1024
ROWS = BATCH * NUM_POS

_INFO = plsc.get_sparse_core_info()
_NC = _INFO.num_cores
_NS = _INFO.num_subcores
_NW = _NC * _NS
_RPW = ROWS // _NW
_CHUNK = 32
_NCHUNK = _RPW // _CHUNK
_LANES = 16
_VECS = EMBED_DIM // _LANES


def _body(tok_hbm, table_hbm, pos_hbm, out_hbm, idx_v, pos_v, buf, sem):
    wid = lax.axis_index("s") * _NC + lax.axis_index("c")
    r0 = wid * _RPW

    pltpu.sync_copy(tok_hbm.at[pl.ds(r0, _RPW)], idx_v)
    pltpu.sync_copy(pos_hbm, pos_v)

    def chunk(k, carry):
        pltpu.async_copy(
            table_hbm.at[idx_v.at[pl.ds(k * _CHUNK, _CHUNK)]], buf, sem
        ).wait()

        def add_row(r, c2):
            p = lax.rem(k * _CHUNK + r, NUM_POS)
            for v in range(_VECS):
                sl = pl.ds(v * _LANES, _LANES)
                buf[r, sl] = buf[r, sl] + pos_v[p, sl]
            return c2

        lax.fori_loop(0, _CHUNK, add_row, 0)

        pltpu.sync_copy(buf, out_hbm.at[pl.ds(r0 + k * _CHUNK, _CHUNK), :])
        return carry

    lax.fori_loop(0, _NCHUNK, chunk, 0)


@jax.jit
def kernel(input_tokens, token_table, pos_table):
    mesh = plsc.VectorSubcoreMesh(core_axis_name="c", subcore_axis_name="s")
    out = pl.kernel(
        _body,
        mesh=mesh,
        out_type=jax.ShapeDtypeStruct((ROWS, EMBED_DIM), jnp.float32),
        scratch_types=[
            pltpu.VMEM((_RPW,), jnp.int32),
            pltpu.VMEM((NUM_POS, EMBED_DIM), jnp.float32),
            pltpu.VMEM((_CHUNK, EMBED_DIM), jnp.float32),
            pltpu.SemaphoreType.DMA,
        ],
    )(input_tokens.astype(jnp.int32).reshape(ROWS), token_table, pos_table)
    return out.reshape(BATCH, NUM_POS, EMBED_DIM)
